# Initial kernel scaffold; baseline (speedup 1.0000x reference)
import jax
import jax.numpy as jnp
from jax.experimental import pallas as pl


def kernel(x, in_proj_w, in_proj_b, out_proj_w, out_proj_b, knowledge_base):
    raise NotImplementedError("write your pallas kernel here")


# speedup vs baseline: 15.0036x; 15.0036x over previous
import functools

import numpy as np
import jax
import jax.numpy as jnp
from jax import lax
from jax.experimental import pallas as pl
from jax.experimental.pallas import tpu as pltpu
from jax.experimental.pallas import tpu_sc as plsc

_M = 64
_N = 2
_K = 16
_SEL_T = 512
_OUT_T = 512
_NW = 32
_CH = 4


def _combine_matrix() -> np.ndarray:
    cm = np.zeros((32, 256), np.float32)
    for a in range(16):
        for b in range(16):
            cm[a, a * 16 + b] = 1.0
            cm[16 + b, a * 16 + b] = 1.0
    return cm


def _select_body(x_ref, w1_ref, b1_ref, cm_ref, wout_ref, iout_ref):
    T = x_ref.shape[0]
    neg = jnp.float32(-1e30)
    big = jnp.float32(1e9)

    h = lax.dot_general(x_ref[...], w1_ref[...], (((1,), (1,)), ((), ())),
                        preferred_element_type=jnp.float32)
    h = h + b1_ref[...]

    lane = lax.broadcasted_iota(jnp.int32, (T, 128), 1).astype(jnp.float32)
    in0 = lane < 64.0

    m0s, a0s, m1s, a1s = [], [], [], []
    hv = h
    for _ in range(_K):
        h0m = jnp.where(in0, hv, neg)
        h1m = jnp.where(in0, neg, hv)
        m0 = jnp.max(h0m, axis=1, keepdims=True)
        m1 = jnp.max(h1m, axis=1, keepdims=True)
        a0 = jnp.min(jnp.where(h0m == m0, lane, big), axis=1, keepdims=True)
        a1 = jnp.min(jnp.where(h1m == m1, lane, big), axis=1, keepdims=True)
        sel = (lane == a0) | (lane == a1)
        hv = jnp.where(sel, neg, hv)
        m0s.append(m0)
        m1s.append(m1)
        a0s.append(a0)
        a1s.append(a1 - 64.0)

    cm = cm_ref[...]
    vcat = jnp.concatenate(m0s + m1s, axis=1)
    fcat = jnp.concatenate(a0s + [a * 64.0 for a in a1s], axis=1)
    cv = lax.dot_general(vcat, cm, (((1,), (0,)), ((), ())),
                         preferred_element_type=jnp.float32,
                         precision=lax.Precision.HIGHEST)
    fv = lax.dot_general(fcat, cm, (((1,), (0,)), ((), ())),
                         preferred_element_type=jnp.float32,
                         precision=lax.Precision.HIGHEST)

    lane256 = lax.broadcasted_iota(jnp.int32, (T, 256), 1).astype(jnp.float32)
    ws, fs = [], []
    for _ in range(_K):
        m = jnp.max(cv, axis=1, keepdims=True)
        pos = jnp.min(jnp.where(cv == m, lane256, big), axis=1, keepdims=True)
        sel = lane256 == pos
        f = jnp.min(jnp.where(sel, fv, big), axis=1, keepdims=True)
        cv = jnp.where(sel, neg, cv)
        ws.append(m)
        fs.append(f)

    wv = jnp.concatenate(ws, axis=1)
    fv16 = jnp.concatenate(fs, axis=1)
    e = jnp.exp(wv - wv[:, 0:1])
    wout_ref[...] = e / jnp.sum(e, axis=1, keepdims=True)
    iout_ref[...] = fv16.astype(jnp.int32)


def _select(x2d, w1, b1, *, interpret=False):
    nt = x2d.shape[0]
    T = _SEL_T
    return pl.pallas_call(
        _select_body,
        grid=(nt // T,),
        in_specs=[
            pl.BlockSpec((T, x2d.shape[1]), lambda i: (i, 0)),
            pl.BlockSpec(w1.shape, lambda i: (0, 0)),
            pl.BlockSpec((1, w1.shape[0]), lambda i: (0, 0)),
            pl.BlockSpec((2 * _K, _K * _K), lambda i: (0, 0)),
        ],
        out_specs=[
            pl.BlockSpec((T, _K), lambda i: (i, 0)),
            pl.BlockSpec((T, _K), lambda i: (i, 0)),
        ],
        out_shape=[
            jax.ShapeDtypeStruct((nt, _K), jnp.float32),
            jax.ShapeDtypeStruct((nt, _K), jnp.int32),
        ],
        interpret=interpret,
    )(x2d, w1, b1.reshape(1, -1), jnp.asarray(_combine_matrix()))


def _sc_combine_body(w_hbm, i_hbm, kb_hbm, out_hbm, idx_v, w_v, rows_v, out_v,
                     sem):
    ntok = out_hbm.shape[0]
    tpw = ntok // _NW
    nchunk = tpw // _CH
    D = kb_hbm.shape[1]
    nj = D // 16
    wid = lax.axis_index("s") * 2 + lax.axis_index("c")
    base_tok = wid * tpw

    def chunk(ci, carry):
        tok0 = base_tok + ci * _CH
        e0 = tok0 * _K
        pltpu.sync_copy(i_hbm.at[pl.ds(e0, _CH * _K)], idx_v)
        pltpu.sync_copy(w_hbm.at[pl.ds(e0, _CH * _K)], w_v)
        pltpu.async_copy(kb_hbm.at[idx_v], rows_v, sem).wait()
        for t in range(_CH):
            w_row = w_v[pl.ds(t * _K, _K)]
            wk = [w_row[k] for k in range(_K)]
            for j in range(nj):
                acc = wk[0] * rows_v[t * _K, pl.ds(j * 16, 16)]
                for k in range(1, _K):
                    acc = acc + wk[k] * rows_v[t * _K + k, pl.ds(j * 16, 16)]
                out_v[t, pl.ds(j * 16, 16)] = acc
        pltpu.sync_copy(out_v, out_hbm.at[pl.ds(tok0, _CH)])
        return carry

    lax.fori_loop(0, nchunk, chunk, 0)


def _combine(w16, i16, kb):
    ntok = w16.shape[0]
    D = kb.shape[1]
    mesh = plsc.VectorSubcoreMesh(core_axis_name="c", subcore_axis_name="s")
    f = pl.kernel(
        _sc_combine_body,
        out_type=jax.ShapeDtypeStruct((ntok, D), jnp.float32),
        mesh=mesh,
        scratch_types=[
            pltpu.VMEM((_CH * _K,), jnp.int32),
            pltpu.VMEM((_CH * _K,), jnp.float32),
            pltpu.VMEM((_CH * _K, D), jnp.float32),
            pltpu.VMEM((_CH, D), jnp.float32),
            pltpu.SemaphoreType.DMA,
        ],
    )
    return f(w16.reshape(-1), i16.reshape(-1), kb)


def _outproj_body(a_ref, w2_ref, b2_ref, y_ref):
    y_ref[...] = lax.dot_general(
        a_ref[...], w2_ref[...], (((1,), (1,)), ((), ())),
        preferred_element_type=jnp.float32) + b2_ref[...]


def _outproj(ans, w2, b2, *, interpret=False):
    nt, D = ans.shape
    E = w2.shape[0]
    T = _OUT_T
    return pl.pallas_call(
        _outproj_body,
        grid=(nt // T,),
        in_specs=[
            pl.BlockSpec((T, D), lambda i: (i, 0)),
            pl.BlockSpec((E, D), lambda i: (0, 0)),
            pl.BlockSpec((1, E), lambda i: (0, 0)),
        ],
        out_specs=pl.BlockSpec((T, E), lambda i: (i, 0)),
        out_shape=jax.ShapeDtypeStruct((nt, E), jnp.float32),
        interpret=interpret,
    )(ans, w2, b2.reshape(1, -1))


def kernel(x, in_proj_w, in_proj_b, out_proj_w, out_proj_b, knowledge_base):
    B, S, E = x.shape
    x2d = x.reshape(B * S, E)
    w16, i16 = _select(x2d, in_proj_w, in_proj_b)
    ans = _combine(w16, i16, knowledge_base)
    y = _outproj(ans, out_proj_w, out_proj_b)
    return y.reshape(B, S, E)


# speedup vs baseline: 22.6385x; 1.5089x over previous
import functools

import numpy as np
import jax
import jax.numpy as jnp
from jax import lax
from jax.experimental import pallas as pl
from jax.experimental.pallas import tpu as pltpu
from jax.experimental.pallas import tpu_sc as plsc

_M = 64
_N = 2
_K = 16
_SEL_T = 512
_OUT_T = 512
_NW = 32
_CH = 4


def _combine_matrix() -> np.ndarray:
    cm = np.zeros((32, 256), np.float32)
    for a in range(16):
        for b in range(16):
            cm[a, a * 16 + b] = 1.0
            cm[16 + b, a * 16 + b] = 1.0
    return cm


def _select_body(x_ref, w1_ref, b1_ref, cm_ref, wout_ref, iout_ref):
    T = x_ref.shape[0]
    neg = jnp.float32(-1e30)
    big = jnp.float32(1e9)

    h = lax.dot_general(x_ref[...], w1_ref[...], (((1,), (1,)), ((), ())),
                        preferred_element_type=jnp.float32)
    h = h + b1_ref[...]

    lane = lax.broadcasted_iota(jnp.int32, (T, 128), 1).astype(jnp.float32)
    in0 = lane < 64.0

    m0s, a0s, m1s, a1s = [], [], [], []
    hv = h
    for _ in range(_K):
        h0m = jnp.where(in0, hv, neg)
        h1m = jnp.where(in0, neg, hv)
        m0 = jnp.max(h0m, axis=1, keepdims=True)
        m1 = jnp.max(h1m, axis=1, keepdims=True)
        a0 = jnp.min(jnp.where(h0m == m0, lane, big), axis=1, keepdims=True)
        a1 = jnp.min(jnp.where(h1m == m1, lane, big), axis=1, keepdims=True)
        sel = (lane == a0) | (lane == a1)
        hv = jnp.where(sel, neg, hv)
        m0s.append(m0)
        m1s.append(m1)
        a0s.append(a0)
        a1s.append(a1 - 64.0)

    cm = cm_ref[...]
    vcat = jnp.concatenate(m0s + m1s, axis=1)
    fcat = jnp.concatenate(a0s + [a * 64.0 for a in a1s], axis=1)
    cv = lax.dot_general(vcat, cm, (((1,), (0,)), ((), ())),
                         preferred_element_type=jnp.float32,
                         precision=lax.Precision.HIGHEST)
    fv = lax.dot_general(fcat, cm, (((1,), (0,)), ((), ())),
                         preferred_element_type=jnp.float32,
                         precision=lax.Precision.HIGHEST)

    lane256 = lax.broadcasted_iota(jnp.int32, (T, 256), 1).astype(jnp.float32)
    ws, fs = [], []
    for _ in range(_K):
        m = jnp.max(cv, axis=1, keepdims=True)
        pos = jnp.min(jnp.where(cv == m, lane256, big), axis=1, keepdims=True)
        sel = lane256 == pos
        f = jnp.min(jnp.where(sel, fv, big), axis=1, keepdims=True)
        cv = jnp.where(sel, neg, cv)
        ws.append(m)
        fs.append(f)

    wv = jnp.concatenate(ws, axis=1)
    fv16 = jnp.concatenate(fs, axis=1)
    e = jnp.exp(wv - wv[:, 0:1])
    wout_ref[...] = e / jnp.sum(e, axis=1, keepdims=True)
    iout_ref[...] = fv16.astype(jnp.int32)


def _select(x2d, w1, b1, *, interpret=False):
    nt = x2d.shape[0]
    T = _SEL_T
    return pl.pallas_call(
        _select_body,
        grid=(nt // T,),
        in_specs=[
            pl.BlockSpec((T, x2d.shape[1]), lambda i: (i, 0)),
            pl.BlockSpec(w1.shape, lambda i: (0, 0)),
            pl.BlockSpec((1, w1.shape[0]), lambda i: (0, 0)),
            pl.BlockSpec((2 * _K, _K * _K), lambda i: (0, 0)),
        ],
        out_specs=[
            pl.BlockSpec((T, _K), lambda i: (i, 0)),
            pl.BlockSpec((T, _K), lambda i: (i, 0)),
        ],
        out_shape=[
            jax.ShapeDtypeStruct((nt, _K), jnp.float32),
            jax.ShapeDtypeStruct((nt, _K), jnp.int32),
        ],
        interpret=interpret,
    )(x2d, w1, b1.reshape(1, -1), jnp.asarray(_combine_matrix()))


def _sc_combine_body(w_hbm, i_hbm, kb_hbm, out_hbm, idx_v, w_v, rows_a, rows_b,
                     out_a, out_b, gsem_a, gsem_b, ssem_a, ssem_b):
    ntok = out_hbm.shape[0]
    tpw = ntok // _NW
    nchunk = tpw // _CH
    npair = nchunk // 2
    D = kb_hbm.shape[1]
    nj = D // 16
    wid = lax.axis_index("s") * 2 + lax.axis_index("c")
    base_tok = wid * tpw

    pltpu.sync_copy(i_hbm.at[pl.ds(wid * nchunk, nchunk)], idx_v)
    pltpu.sync_copy(w_hbm.at[pl.ds(base_tok * _K, tpw * _K)], w_v)

    def gather(ci, rows, gsem):
        pltpu.async_copy(kb_hbm.at[idx_v.at[ci]], rows, gsem)

    def gather_wait(rows, gsem):
        pltpu.make_async_copy(kb_hbm.at[idx_v.at[0]], rows, gsem).wait()

    def compute_store(ci, rows, out, ssem):
        for t in range(_CH):
            w_row = w_v[pl.ds(ci * _CH * _K + t * _K, _K)]
            wk = [w_row[k] for k in range(_K)]
            for j in range(nj):
                acc = wk[0] * rows[t * _K, pl.ds(j * 16, 16)]
                for k in range(1, _K):
                    acc = acc + wk[k] * rows[t * _K + k, pl.ds(j * 16, 16)]
                out[t, pl.ds(j * 16, 16)] = acc
        pltpu.async_copy(out, out_hbm.at[pl.ds(base_tok + ci * _CH, _CH)], ssem)

    def store_wait(out, ssem):
        pltpu.make_async_copy(out, out_hbm.at[pl.ds(base_tok, _CH)], ssem).wait()

    gather(0, rows_a, gsem_a)
    gather(1, rows_b, gsem_b)

    def pair(p, carry):
        c0 = 2 * p
        gather_wait(rows_a, gsem_a)

        @pl.when(p > 0)
        def _():
            store_wait(out_a, ssem_a)

        compute_store(c0, rows_a, out_a, ssem_a)

        @pl.when(p < npair - 1)
        def _():
            gather(c0 + 2, rows_a, gsem_a)

        gather_wait(rows_b, gsem_b)

        @pl.when(p > 0)
        def _():
            store_wait(out_b, ssem_b)

        compute_store(c0 + 1, rows_b, out_b, ssem_b)

        @pl.when(p < npair - 1)
        def _():
            gather(c0 + 3, rows_b, gsem_b)

        return carry

    lax.fori_loop(0, npair, pair, 0)
    store_wait(out_a, ssem_a)
    store_wait(out_b, ssem_b)


def _combine(w16, i16, kb):
    ntok = w16.shape[0]
    D = kb.shape[1]
    tpw = ntok // _NW
    nchunk = tpw // _CH
    mesh = plsc.VectorSubcoreMesh(core_axis_name="c", subcore_axis_name="s")
    f = pl.kernel(
        _sc_combine_body,
        out_type=jax.ShapeDtypeStruct((ntok, D), jnp.float32),
        mesh=mesh,
        scratch_types=[
            pltpu.VMEM((nchunk, _CH * _K), jnp.int32),
            pltpu.VMEM((tpw * _K,), jnp.float32),
            pltpu.VMEM((_CH * _K, D), jnp.float32),
            pltpu.VMEM((_CH * _K, D), jnp.float32),
            pltpu.VMEM((_CH, D), jnp.float32),
            pltpu.VMEM((_CH, D), jnp.float32),
            pltpu.SemaphoreType.DMA,
            pltpu.SemaphoreType.DMA,
            pltpu.SemaphoreType.DMA,
            pltpu.SemaphoreType.DMA,
        ],
    )
    return f(w16.reshape(-1), i16.reshape(ntok // _CH, _CH * _K), kb)


def _outproj_body(a_ref, w2_ref, b2_ref, y_ref):
    y_ref[...] = lax.dot_general(
        a_ref[...], w2_ref[...], (((1,), (1,)), ((), ())),
        preferred_element_type=jnp.float32) + b2_ref[...]


def _outproj(ans, w2, b2, *, interpret=False):
    nt, D = ans.shape
    E = w2.shape[0]
    T = _OUT_T
    return pl.pallas_call(
        _outproj_body,
        grid=(nt // T,),
        in_specs=[
            pl.BlockSpec((T, D), lambda i: (i, 0)),
            pl.BlockSpec((E, D), lambda i: (0, 0)),
            pl.BlockSpec((1, E), lambda i: (0, 0)),
        ],
        out_specs=pl.BlockSpec((T, E), lambda i: (i, 0)),
        out_shape=jax.ShapeDtypeStruct((nt, E), jnp.float32),
        interpret=interpret,
    )(ans, w2, b2.reshape(1, -1))


def kernel(x, in_proj_w, in_proj_b, out_proj_w, out_proj_b, knowledge_base):
    B, S, E = x.shape
    x2d = x.reshape(B * S, E)
    w16, i16 = _select(x2d, in_proj_w, in_proj_b)
    ans = _combine(w16, i16, knowledge_base)
    y = _outproj(ans, out_proj_w, out_proj_b)
    return y.reshape(B, S, E)


# speedup vs baseline: 22.7294x; 1.0040x over previous
import functools

import numpy as np
import jax
import jax.numpy as jnp
from jax import lax
from jax.experimental import pallas as pl
from jax.experimental.pallas import tpu as pltpu
from jax.experimental.pallas import tpu_sc as plsc

_M = 64
_N = 2
_K = 16
_SEL_T = 512
_OUT_T = 512
_NW = 32
_CH = 4


_PAIRS = [(a, b) for a in range(_K) for b in range(_K) if (a + 1) * (b + 1) <= _K]
_NC = 64


def _combine_matrix() -> np.ndarray:
    cm = np.zeros((2 * _K + 1, _NC), np.float32)
    for q, (a, b) in enumerate(_PAIRS):
        cm[a, q] = 1.0
        cm[_K + b, q] = 1.0
    for q in range(len(_PAIRS), _NC):
        cm[2 * _K, q] = 1.0
    return cm


def _select_body(x_ref, w1_ref, b1_ref, cm_ref, wout_ref, iout_ref):
    T = x_ref.shape[0]
    neg = jnp.float32(-1e30)
    big = jnp.float32(1e9)

    h = lax.dot_general(x_ref[...], w1_ref[...], (((1,), (1,)), ((), ())),
                        preferred_element_type=jnp.float32)
    h = h + b1_ref[...]

    lane = lax.broadcasted_iota(jnp.int32, (T, 128), 1).astype(jnp.float32)
    in0 = lane < 64.0

    m0s, a0s, m1s, a1s = [], [], [], []
    hv = h
    for _ in range(_K):
        h0m = jnp.where(in0, hv, neg)
        h1m = jnp.where(in0, neg, hv)
        m0 = jnp.max(h0m, axis=1, keepdims=True)
        m1 = jnp.max(h1m, axis=1, keepdims=True)
        a0 = jnp.min(jnp.where(h0m == m0, lane, big), axis=1, keepdims=True)
        a1 = jnp.min(jnp.where(h1m == m1, lane, big), axis=1, keepdims=True)
        sel = (lane == a0) | (lane == a1)
        hv = jnp.where(sel, neg, hv)
        m0s.append(m0)
        m1s.append(m1)
        a0s.append(a0)
        a1s.append(a1 - 64.0)

    cm = cm_ref[...]
    negcol = jnp.full((T, 1), neg, jnp.float32)
    zerocol = jnp.zeros((T, 1), jnp.float32)
    vcat = jnp.concatenate(m0s + m1s + [negcol], axis=1)
    fcat = jnp.concatenate(a0s + [a * 64.0 for a in a1s] + [zerocol],
                           axis=1)
    cv = lax.dot_general(vcat, cm, (((1,), (0,)), ((), ())),
                         preferred_element_type=jnp.float32,
                         precision=lax.Precision.HIGHEST)
    fv = lax.dot_general(fcat, cm, (((1,), (0,)), ((), ())),
                         preferred_element_type=jnp.float32,
                         precision=lax.Precision.HIGHEST)

    lane256 = lax.broadcasted_iota(jnp.int32, (T, _NC), 1).astype(jnp.float32)
    ws, fs = [], []
    for _ in range(_K):
        m = jnp.max(cv, axis=1, keepdims=True)
        pos = jnp.min(jnp.where(cv == m, lane256, big), axis=1, keepdims=True)
        sel = lane256 == pos
        f = jnp.min(jnp.where(sel, fv, big), axis=1, keepdims=True)
        cv = jnp.where(sel, neg, cv)
        ws.append(m)
        fs.append(f)

    wv = jnp.concatenate(ws, axis=1)
    fv16 = jnp.concatenate(fs, axis=1)
    e = jnp.exp(wv - wv[:, 0:1])
    wout_ref[...] = e / jnp.sum(e, axis=1, keepdims=True)
    iout_ref[...] = fv16.astype(jnp.int32)


def _select(x2d, w1, b1, *, interpret=False):
    nt = x2d.shape[0]
    T = _SEL_T
    return pl.pallas_call(
        _select_body,
        grid=(nt // T,),
        in_specs=[
            pl.BlockSpec((T, x2d.shape[1]), lambda i: (i, 0)),
            pl.BlockSpec(w1.shape, lambda i: (0, 0)),
            pl.BlockSpec((1, w1.shape[0]), lambda i: (0, 0)),
            pl.BlockSpec((2 * _K + 1, _NC), lambda i: (0, 0)),
        ],
        out_specs=[
            pl.BlockSpec((T, _K), lambda i: (i, 0)),
            pl.BlockSpec((T, _K), lambda i: (i, 0)),
        ],
        out_shape=[
            jax.ShapeDtypeStruct((nt, _K), jnp.float32),
            jax.ShapeDtypeStruct((nt, _K), jnp.int32),
        ],
        interpret=interpret,
    )(x2d, w1, b1.reshape(1, -1), jnp.asarray(_combine_matrix()))


def _sc_combine_body(w_hbm, i_hbm, kb_hbm, out_hbm, idx_v, w_v, rows_a, rows_b,
                     out_a, out_b, gsem_a, gsem_b, ssem_a, ssem_b):
    ntok = out_hbm.shape[0]
    tpw = ntok // _NW
    nchunk = tpw // _CH
    npair = nchunk // 2
    D = kb_hbm.shape[1]
    nj = D // 16
    wid = lax.axis_index("s") * 2 + lax.axis_index("c")
    base_tok = wid * tpw

    pltpu.sync_copy(i_hbm.at[pl.ds(wid * nchunk, nchunk)], idx_v)
    pltpu.sync_copy(w_hbm.at[pl.ds(base_tok * _K, tpw * _K)], w_v)

    def gather(ci, rows, gsem):
        pltpu.async_copy(kb_hbm.at[idx_v.at[ci]], rows, gsem)

    def gather_wait(rows, gsem):
        pltpu.make_async_copy(kb_hbm.at[idx_v.at[0]], rows, gsem).wait()

    def compute_store(ci, rows, out, ssem):
        for t in range(_CH):
            w_row = w_v[pl.ds(ci * _CH * _K + t * _K, _K)]
            wk = [w_row[k] for k in range(_K)]
            for j in range(nj):
                acc = wk[0] * rows[t * _K, pl.ds(j * 16, 16)]
                for k in range(1, _K):
                    acc = acc + wk[k] * rows[t * _K + k, pl.ds(j * 16, 16)]
                out[t, pl.ds(j * 16, 16)] = acc
        pltpu.async_copy(out, out_hbm.at[pl.ds(base_tok + ci * _CH, _CH)], ssem)

    def store_wait(out, ssem):
        pltpu.make_async_copy(out, out_hbm.at[pl.ds(base_tok, _CH)], ssem).wait()

    gather(0, rows_a, gsem_a)
    gather(1, rows_b, gsem_b)

    def pair(p, carry):
        c0 = 2 * p
        gather_wait(rows_a, gsem_a)

        @pl.when(p > 0)
        def _():
            store_wait(out_a, ssem_a)

        compute_store(c0, rows_a, out_a, ssem_a)

        @pl.when(p < npair - 1)
        def _():
            gather(c0 + 2, rows_a, gsem_a)

        gather_wait(rows_b, gsem_b)

        @pl.when(p > 0)
        def _():
            store_wait(out_b, ssem_b)

        compute_store(c0 + 1, rows_b, out_b, ssem_b)

        @pl.when(p < npair - 1)
        def _():
            gather(c0 + 3, rows_b, gsem_b)

        return carry

    lax.fori_loop(0, npair, pair, 0)
    store_wait(out_a, ssem_a)
    store_wait(out_b, ssem_b)


def _combine(w16, i16, kb):
    ntok = w16.shape[0]
    D = kb.shape[1]
    tpw = ntok // _NW
    nchunk = tpw // _CH
    mesh = plsc.VectorSubcoreMesh(core_axis_name="c", subcore_axis_name="s")
    f = pl.kernel(
        _sc_combine_body,
        out_type=jax.ShapeDtypeStruct((ntok, D), jnp.float32),
        mesh=mesh,
        scratch_types=[
            pltpu.VMEM((nchunk, _CH * _K), jnp.int32),
            pltpu.VMEM((tpw * _K,), jnp.float32),
            pltpu.VMEM((_CH * _K, D), jnp.float32),
            pltpu.VMEM((_CH * _K, D), jnp.float32),
            pltpu.VMEM((_CH, D), jnp.float32),
            pltpu.VMEM((_CH, D), jnp.float32),
            pltpu.SemaphoreType.DMA,
            pltpu.SemaphoreType.DMA,
            pltpu.SemaphoreType.DMA,
            pltpu.SemaphoreType.DMA,
        ],
    )
    return f(w16.reshape(-1), i16.reshape(ntok // _CH, _CH * _K), kb)


def _outproj_body(a_ref, w2_ref, b2_ref, y_ref):
    y_ref[...] = lax.dot_general(
        a_ref[...], w2_ref[...], (((1,), (1,)), ((), ())),
        preferred_element_type=jnp.float32) + b2_ref[...]


def _outproj(ans, w2, b2, *, interpret=False):
    nt, D = ans.shape
    E = w2.shape[0]
    T = _OUT_T
    return pl.pallas_call(
        _outproj_body,
        grid=(nt // T,),
        in_specs=[
            pl.BlockSpec((T, D), lambda i: (i, 0)),
            pl.BlockSpec((E, D), lambda i: (0, 0)),
            pl.BlockSpec((1, E), lambda i: (0, 0)),
        ],
        out_specs=pl.BlockSpec((T, E), lambda i: (i, 0)),
        out_shape=jax.ShapeDtypeStruct((nt, E), jnp.float32),
        interpret=interpret,
    )(ans, w2, b2.reshape(1, -1))


def kernel(x, in_proj_w, in_proj_b, out_proj_w, out_proj_b, knowledge_base):
    B, S, E = x.shape
    x2d = x.reshape(B * S, E)
    w16, i16 = _select(x2d, in_proj_w, in_proj_b)
    ans = _combine(w16, i16, knowledge_base)
    y = _outproj(ans, out_proj_w, out_proj_b)
    return y.reshape(B, S, E)


# speedup vs baseline: 24.0232x; 1.0569x over previous
import functools

import numpy as np
import jax
import jax.numpy as jnp
from jax import lax
from jax.experimental import pallas as pl
from jax.experimental.pallas import tpu as pltpu
from jax.experimental.pallas import tpu_sc as plsc

_M = 64
_N = 2
_K = 16
_SEL_T = 512
_OUT_T = 512
_NW = 32
_CH = 4


_PAIRS = [(a, b) for a in range(_K) for b in range(_K) if (a + 1) * (b + 1) <= _K]
_NC = 64


def _combine_matrix() -> np.ndarray:
    cm = np.zeros((2 * _K + 1, _NC), np.float32)
    for q, (a, b) in enumerate(_PAIRS):
        cm[a, q] = 1.0
        cm[_K + b, q] = 1.0
    for q in range(len(_PAIRS), _NC):
        cm[2 * _K, q] = 1.0
    return cm


def _select_body(x_ref, w1_ref, b1_ref, cm_ref, wout_ref, iout_ref):
    T = x_ref.shape[0]
    neg = jnp.float32(-1e30)
    big = jnp.float32(1e9)

    h = lax.dot_general(x_ref[...], w1_ref[...], (((1,), (1,)), ((), ())),
                        preferred_element_type=jnp.float32)
    h = h + b1_ref[...]

    lane = lax.broadcasted_iota(jnp.int32, (T, 128), 1).astype(jnp.float32)
    in0 = lane < 64.0

    m0s, a0s, m1s, a1s = [], [], [], []
    hv = h
    for _ in range(_K):
        h0m = jnp.where(in0, hv, neg)
        h1m = jnp.where(in0, neg, hv)
        m0 = jnp.max(h0m, axis=1, keepdims=True)
        m1 = jnp.max(h1m, axis=1, keepdims=True)
        eq0 = h0m == m0
        eq1 = h1m == m1
        hv = jnp.where(eq0 | eq1, neg, hv)
        a0 = jnp.min(jnp.where(eq0, lane, big), axis=1, keepdims=True)
        a1 = jnp.min(jnp.where(eq1, lane, big), axis=1, keepdims=True)
        m0s.append(m0)
        m1s.append(m1)
        a0s.append(a0)
        a1s.append(a1 - 64.0)

    cm = cm_ref[...]
    negcol = jnp.full((T, 1), neg, jnp.float32)
    zerocol = jnp.zeros((T, 1), jnp.float32)
    vcat = jnp.concatenate(m0s + m1s + [negcol], axis=1)
    fcat = jnp.concatenate(a0s + [a * 64.0 for a in a1s] + [zerocol],
                           axis=1)
    cv = lax.dot_general(vcat, cm, (((1,), (0,)), ((), ())),
                         preferred_element_type=jnp.float32,
                         precision=lax.Precision.HIGHEST)
    fv = lax.dot_general(fcat, cm, (((1,), (0,)), ((), ())),
                         preferred_element_type=jnp.float32,
                         precision=lax.Precision.HIGHEST)

    ws, fs = [], []
    for _ in range(_K):
        m = jnp.max(cv, axis=1, keepdims=True)
        eq = cv == m
        cv = jnp.where(eq, neg, cv)
        f = jnp.min(jnp.where(eq, fv, big), axis=1, keepdims=True)
        ws.append(m)
        fs.append(f)

    wv = jnp.concatenate(ws, axis=1)
    fv16 = jnp.concatenate(fs, axis=1)
    e = jnp.exp(wv - wv[:, 0:1])
    wout_ref[...] = e / jnp.sum(e, axis=1, keepdims=True)
    iout_ref[...] = fv16.astype(jnp.int32)


def _select(x2d, w1, b1, *, interpret=False):
    nt = x2d.shape[0]
    T = _SEL_T
    return pl.pallas_call(
        _select_body,
        grid=(nt // T,),
        in_specs=[
            pl.BlockSpec((T, x2d.shape[1]), lambda i: (i, 0)),
            pl.BlockSpec(w1.shape, lambda i: (0, 0)),
            pl.BlockSpec((1, w1.shape[0]), lambda i: (0, 0)),
            pl.BlockSpec((2 * _K + 1, _NC), lambda i: (0, 0)),
        ],
        out_specs=[
            pl.BlockSpec((T, _K), lambda i: (i, 0)),
            pl.BlockSpec((T, _K), lambda i: (i, 0)),
        ],
        out_shape=[
            jax.ShapeDtypeStruct((nt, _K), jnp.float32),
            jax.ShapeDtypeStruct((nt, _K), jnp.int32),
        ],
        interpret=interpret,
    )(x2d, w1, b1.reshape(1, -1), jnp.asarray(_combine_matrix()))


def _sc_combine_body(w_hbm, i_hbm, kb_hbm, out_hbm, idx_v, w_v, rows_a, rows_b,
                     out_a, out_b, gsem_a, gsem_b, ssem_a, ssem_b):
    ntok = out_hbm.shape[0]
    tpw = ntok // _NW
    nchunk = tpw // _CH
    npair = nchunk // 2
    D = kb_hbm.shape[1]
    nj = D // 16
    wid = lax.axis_index("s") * 2 + lax.axis_index("c")
    base_tok = wid * tpw

    pltpu.sync_copy(i_hbm.at[pl.ds(wid * nchunk, nchunk)], idx_v)
    pltpu.sync_copy(w_hbm.at[pl.ds(base_tok * _K, tpw * _K)], w_v)

    def gather(ci, rows, gsem):
        pltpu.async_copy(kb_hbm.at[idx_v.at[ci]], rows, gsem)

    def gather_wait(rows, gsem):
        pltpu.make_async_copy(kb_hbm.at[idx_v.at[0]], rows, gsem).wait()

    def compute_store(ci, rows, out, ssem):
        for t in range(_CH):
            w_row = w_v[pl.ds(ci * _CH * _K + t * _K, _K)]
            wk = [w_row[k] for k in range(_K)]
            for j in range(nj):
                acc = wk[0] * rows[t * _K, pl.ds(j * 16, 16)]
                for k in range(1, _K):
                    acc = acc + wk[k] * rows[t * _K + k, pl.ds(j * 16, 16)]
                out[t, pl.ds(j * 16, 16)] = acc
        pltpu.async_copy(out, out_hbm.at[pl.ds(base_tok + ci * _CH, _CH)], ssem)

    def store_wait(out, ssem):
        pltpu.make_async_copy(out, out_hbm.at[pl.ds(base_tok, _CH)], ssem).wait()

    gather(0, rows_a, gsem_a)
    gather(1, rows_b, gsem_b)

    def pair(p, carry):
        c0 = 2 * p
        gather_wait(rows_a, gsem_a)

        @pl.when(p > 0)
        def _():
            store_wait(out_a, ssem_a)

        compute_store(c0, rows_a, out_a, ssem_a)

        @pl.when(p < npair - 1)
        def _():
            gather(c0 + 2, rows_a, gsem_a)

        gather_wait(rows_b, gsem_b)

        @pl.when(p > 0)
        def _():
            store_wait(out_b, ssem_b)

        compute_store(c0 + 1, rows_b, out_b, ssem_b)

        @pl.when(p < npair - 1)
        def _():
            gather(c0 + 3, rows_b, gsem_b)

        return carry

    lax.fori_loop(0, npair, pair, 0)
    store_wait(out_a, ssem_a)
    store_wait(out_b, ssem_b)


def _combine(w16, i16, kb):
    ntok = w16.shape[0]
    D = kb.shape[1]
    tpw = ntok // _NW
    nchunk = tpw // _CH
    mesh = plsc.VectorSubcoreMesh(core_axis_name="c", subcore_axis_name="s")
    f = pl.kernel(
        _sc_combine_body,
        out_type=jax.ShapeDtypeStruct((ntok, D), jnp.float32),
        mesh=mesh,
        scratch_types=[
            pltpu.VMEM((nchunk, _CH * _K), jnp.int32),
            pltpu.VMEM((tpw * _K,), jnp.float32),
            pltpu.VMEM((_CH * _K, D), jnp.float32),
            pltpu.VMEM((_CH * _K, D), jnp.float32),
            pltpu.VMEM((_CH, D), jnp.float32),
            pltpu.VMEM((_CH, D), jnp.float32),
            pltpu.SemaphoreType.DMA,
            pltpu.SemaphoreType.DMA,
            pltpu.SemaphoreType.DMA,
            pltpu.SemaphoreType.DMA,
        ],
    )
    return f(w16.reshape(-1), i16.reshape(ntok // _CH, _CH * _K), kb)


def _outproj_body(a_ref, w2_ref, b2_ref, y_ref):
    y_ref[...] = lax.dot_general(
        a_ref[...], w2_ref[...], (((1,), (1,)), ((), ())),
        preferred_element_type=jnp.float32) + b2_ref[...]


def _outproj(ans, w2, b2, *, interpret=False):
    nt, D = ans.shape
    E = w2.shape[0]
    T = _OUT_T
    return pl.pallas_call(
        _outproj_body,
        grid=(nt // T,),
        in_specs=[
            pl.BlockSpec((T, D), lambda i: (i, 0)),
            pl.BlockSpec((E, D), lambda i: (0, 0)),
            pl.BlockSpec((1, E), lambda i: (0, 0)),
        ],
        out_specs=pl.BlockSpec((T, E), lambda i: (i, 0)),
        out_shape=jax.ShapeDtypeStruct((nt, E), jnp.float32),
        interpret=interpret,
    )(ans, w2, b2.reshape(1, -1))


def kernel(x, in_proj_w, in_proj_b, out_proj_w, out_proj_b, knowledge_base):
    B, S, E = x.shape
    x2d = x.reshape(B * S, E)
    w16, i16 = _select(x2d, in_proj_w, in_proj_b)
    ans = _combine(w16, i16, knowledge_base)
    y = _outproj(ans, out_proj_w, out_proj_b)
    return y.reshape(B, S, E)


# speedup vs baseline: 25.3996x; 1.0573x over previous
import functools

import numpy as np
import jax
import jax.numpy as jnp
from jax import lax
from jax.experimental import pallas as pl
from jax.experimental.pallas import tpu as pltpu
from jax.experimental.pallas import tpu_sc as plsc

_M = 64
_N = 2
_K = 16
_SEL_T = 512
_OUT_T = 512
_NW = 32
_CH = 4


_PAIRS = [(a, b) for a in range(_K) for b in range(_K) if (a + 1) * (b + 1) <= _K]
_NC = 64


def _combine_matrix() -> np.ndarray:
    cm = np.zeros((2 * _K + 1, _NC), np.float32)
    for q, (a, b) in enumerate(_PAIRS):
        cm[a, q] = 1.0
        cm[_K + b, q] = 1.0
    for q in range(len(_PAIRS), _NC):
        cm[2 * _K, q] = 1.0
    return cm


def _select_body(x_ref, w1_ref, b1_ref, cm_ref, wout_ref, iout_ref):
    T = x_ref.shape[0]
    neg = jnp.float32(-1e30)
    big = jnp.float32(1e9)

    h = lax.dot_general(x_ref[...], w1_ref[...], (((1,), (1,)), ((), ())),
                        preferred_element_type=jnp.float32)
    h = h + b1_ref[...]

    lane = lax.broadcasted_iota(jnp.int32, (T, 128), 1).astype(jnp.float32)
    in0 = lane < 64.0

    m0s, a0s, m1s, a1s = [], [], [], []
    hv = h
    for _ in range(_K):
        h0m = jnp.where(in0, hv, neg)
        h1m = jnp.where(in0, neg, hv)
        m0 = jnp.max(h0m, axis=1, keepdims=True)
        m1 = jnp.max(h1m, axis=1, keepdims=True)
        eq0 = h0m == m0
        eq1 = h1m == m1
        hv = jnp.where(eq0 | eq1, neg, hv)
        a0 = jnp.min(jnp.where(eq0, lane, big), axis=1, keepdims=True)
        a1 = jnp.min(jnp.where(eq1, lane, big), axis=1, keepdims=True)
        m0s.append(m0)
        m1s.append(m1)
        a0s.append(a0)
        a1s.append(a1 - 64.0)

    cm = cm_ref[...]
    negcol = jnp.full((T, 1), neg, jnp.float32)
    zerocol = jnp.zeros((T, 1), jnp.float32)
    vcat = jnp.concatenate(m0s + m1s + [negcol], axis=1)
    fcat = jnp.concatenate(a0s + [a * 64.0 for a in a1s] + [zerocol],
                           axis=1)
    cv = lax.dot_general(vcat, cm, (((1,), (0,)), ((), ())),
                         preferred_element_type=jnp.float32,
                         precision=lax.Precision.HIGHEST)
    fv = lax.dot_general(fcat, cm, (((1,), (0,)), ((), ())),
                         preferred_element_type=jnp.float32,
                         precision=lax.Precision.HIGHEST)

    ws, fs = [], []
    for _ in range(_K):
        m = jnp.max(cv, axis=1, keepdims=True)
        eq = cv == m
        cv = jnp.where(eq, neg, cv)
        f = jnp.min(jnp.where(eq, fv, big), axis=1, keepdims=True)
        ws.append(m)
        fs.append(f)

    wv = jnp.concatenate(ws, axis=1)
    fv16 = jnp.concatenate(fs, axis=1)
    e = jnp.exp(wv - wv[:, 0:1])
    wout_ref[...] = e / jnp.sum(e, axis=1, keepdims=True)
    iout_ref[...] = fv16.astype(jnp.int32)


def _select(x2d, w1, b1, *, interpret=False):
    nt = x2d.shape[0]
    T = _SEL_T
    return pl.pallas_call(
        _select_body,
        grid=(nt // T,),
        in_specs=[
            pl.BlockSpec((T, x2d.shape[1]), lambda i: (i, 0)),
            pl.BlockSpec(w1.shape, lambda i: (0, 0)),
            pl.BlockSpec((1, w1.shape[0]), lambda i: (0, 0)),
            pl.BlockSpec((2 * _K + 1, _NC), lambda i: (0, 0)),
        ],
        out_specs=[
            pl.BlockSpec((T, _K), lambda i: (i, 0)),
            pl.BlockSpec((T, _K), lambda i: (i, 0)),
        ],
        out_shape=[
            jax.ShapeDtypeStruct((nt, _K), jnp.float32),
            jax.ShapeDtypeStruct((nt, _K), jnp.int32),
        ],
        interpret=interpret,
    )(x2d, w1, b1.reshape(1, -1), jnp.asarray(_combine_matrix()))


def _sc_combine_body(w_hbm, i_hbm, kb_hbm, out_hbm, idx_v, w_v, rows_a, rows_b,
                     out_a, out_b, gsem_a, gsem_b, ssem_a, ssem_b):
    ntok = out_hbm.shape[0]
    tpw = ntok // _NW
    nchunk = tpw // _CH
    npair = nchunk // 2
    D = kb_hbm.shape[1]
    nj = D // 16
    wid = lax.axis_index("s") * 2 + lax.axis_index("c")
    base_tok = wid * tpw

    pltpu.sync_copy(i_hbm.at[pl.ds(wid * nchunk, nchunk)], idx_v)
    pltpu.sync_copy(w_hbm.at[pl.ds(base_tok * _K, tpw * _K)], w_v)

    def gather(ci, rows, gsem):
        pltpu.async_copy(kb_hbm.at[idx_v.at[ci]], rows, gsem)

    def gather_wait(rows, gsem):
        pltpu.make_async_copy(kb_hbm.at[idx_v.at[0]], rows, gsem).wait()

    def compute_store(ci, rows, out, ssem):
        for t in range(_CH):
            w_row = w_v[pl.ds(ci * _CH * _K + t * _K, _K)]
            wk = [w_row[k] for k in range(_K)]
            for j in range(nj):
                acc = wk[0] * rows[t * _K, pl.ds(j * 16, 16)]
                for k in range(1, _K):
                    acc = acc + wk[k] * rows[t * _K + k, pl.ds(j * 16, 16)]
                out[t, pl.ds(j * 16, 16)] = acc
        pltpu.async_copy(out, out_hbm.at[pl.ds(base_tok + ci * _CH, _CH)], ssem)

    def store_wait(out, ssem):
        pltpu.make_async_copy(out, out_hbm.at[pl.ds(base_tok, _CH)], ssem).wait()

    gather(0, rows_a, gsem_a)
    gather(1, rows_b, gsem_b)

    def pair(p, carry):
        c0 = 2 * p
        gather_wait(rows_a, gsem_a)

        @pl.when(p > 0)
        def _():
            store_wait(out_a, ssem_a)

        compute_store(c0, rows_a, out_a, ssem_a)

        @pl.when(p < npair - 1)
        def _():
            gather(c0 + 2, rows_a, gsem_a)

        gather_wait(rows_b, gsem_b)

        @pl.when(p > 0)
        def _():
            store_wait(out_b, ssem_b)

        compute_store(c0 + 1, rows_b, out_b, ssem_b)

        @pl.when(p < npair - 1)
        def _():
            gather(c0 + 3, rows_b, gsem_b)

        return carry

    lax.fori_loop(0, npair, pair, 0)
    store_wait(out_a, ssem_a)
    store_wait(out_b, ssem_b)


def _combine(w16, i16, kb):
    ntok = w16.shape[0]
    D = kb.shape[1]
    tpw = ntok // _NW
    nchunk = tpw // _CH
    mesh = plsc.VectorSubcoreMesh(core_axis_name="c", subcore_axis_name="s")
    f = pl.kernel(
        _sc_combine_body,
        out_type=jax.ShapeDtypeStruct((ntok, D), jnp.float32),
        mesh=mesh,
        scratch_types=[
            pltpu.VMEM((nchunk, _CH * _K), jnp.int32),
            pltpu.VMEM((tpw * _K,), jnp.float32),
            pltpu.VMEM((_CH * _K, D), jnp.float32),
            pltpu.VMEM((_CH * _K, D), jnp.float32),
            pltpu.VMEM((_CH, D), jnp.float32),
            pltpu.VMEM((_CH, D), jnp.float32),
            pltpu.SemaphoreType.DMA,
            pltpu.SemaphoreType.DMA,
            pltpu.SemaphoreType.DMA,
            pltpu.SemaphoreType.DMA,
        ],
    )
    return f(w16.reshape(-1), i16.reshape(ntok // _CH, _CH * _K), kb)


def _outproj_body(a_ref, w2_ref, b2_ref, y_ref):
    y_ref[...] = lax.dot_general(
        a_ref[...], w2_ref[...], (((1,), (1,)), ((), ())),
        preferred_element_type=jnp.float32) + b2_ref[...]


def _outproj(ans, w2, b2, *, interpret=False):
    nt, D = ans.shape
    E = w2.shape[0]
    T = _OUT_T
    return pl.pallas_call(
        _outproj_body,
        grid=(nt // T,),
        in_specs=[
            pl.BlockSpec((T, D), lambda i: (i, 0)),
            pl.BlockSpec((E, D), lambda i: (0, 0)),
            pl.BlockSpec((1, E), lambda i: (0, 0)),
        ],
        out_specs=pl.BlockSpec((T, E), lambda i: (i, 0)),
        out_shape=jax.ShapeDtypeStruct((nt, E), jnp.float32),
        interpret=interpret,
    )(ans, w2, b2.reshape(1, -1))


def kernel(x, in_proj_w, in_proj_b, out_proj_w, out_proj_b, knowledge_base):
    B, S, E = x.shape
    x2d = x.reshape(B * S, E)
    nh = (B * S) // 2
    ys = []
    sel = [_select(x2d[i * nh:(i + 1) * nh], in_proj_w, in_proj_b)
           for i in range(2)]
    for i in range(2):
        w16, i16 = sel[i]
        ans = _combine(w16, i16, knowledge_base)
        ys.append(_outproj(ans, out_proj_w, out_proj_b))
    return jnp.concatenate(ys, axis=0).reshape(B, S, E)


# speedup vs baseline: 25.5441x; 1.0057x over previous
import functools

import numpy as np
import jax
import jax.numpy as jnp
from jax import lax
from jax.experimental import pallas as pl
from jax.experimental.pallas import tpu as pltpu
from jax.experimental.pallas import tpu_sc as plsc

_M = 64
_N = 2
_K = 16
_SEL_T = 512
_OUT_T = 512
_NW = 32
_CH = 4


_PAIRS = [(a, b) for a in range(_K) for b in range(_K) if (a + 1) * (b + 1) <= _K]
_NC = 64


def _combine_matrix() -> np.ndarray:
    cm = np.zeros((2 * _K + 1, _NC), np.float32)
    for q, (a, b) in enumerate(_PAIRS):
        cm[a, q] = 1.0
        cm[_K + b, q] = 1.0
    for q in range(len(_PAIRS), _NC):
        cm[2 * _K, q] = 1.0
    return cm


def _select_body(x_ref, w1_ref, b1_ref, cm_ref, wout_ref, iout_ref):
    T = x_ref.shape[0]
    neg = jnp.float32(-1e30)
    big = jnp.float32(1e9)

    h = lax.dot_general(x_ref[...], w1_ref[...], (((1,), (1,)), ((), ())),
                        preferred_element_type=jnp.float32)
    h = h + b1_ref[...]

    lane = lax.broadcasted_iota(jnp.int32, (T, 128), 1).astype(jnp.float32)
    in0 = lane < 64.0

    m0s, a0s, m1s, a1s = [], [], [], []
    hv = h
    for _ in range(_K):
        h0m = jnp.where(in0, hv, neg)
        h1m = jnp.where(in0, neg, hv)
        m0 = jnp.max(h0m, axis=1, keepdims=True)
        m1 = jnp.max(h1m, axis=1, keepdims=True)
        eq0 = h0m == m0
        eq1 = h1m == m1
        hv = jnp.where(eq0 | eq1, neg, hv)
        a0 = jnp.min(jnp.where(eq0, lane, big), axis=1, keepdims=True)
        a1 = jnp.min(jnp.where(eq1, lane, big), axis=1, keepdims=True)
        m0s.append(m0)
        m1s.append(m1)
        a0s.append(a0)
        a1s.append(a1 - 64.0)

    cm = cm_ref[...]
    negcol = jnp.full((T, 1), neg, jnp.float32)
    zerocol = jnp.zeros((T, 1), jnp.float32)
    vcat = jnp.concatenate(m0s + m1s + [negcol], axis=1)
    fcat = jnp.concatenate(a0s + [a * 64.0 for a in a1s] + [zerocol],
                           axis=1)
    cv = lax.dot_general(vcat, cm, (((1,), (0,)), ((), ())),
                         preferred_element_type=jnp.float32,
                         precision=lax.Precision.HIGHEST)
    fv = lax.dot_general(fcat, cm, (((1,), (0,)), ((), ())),
                         preferred_element_type=jnp.float32,
                         precision=lax.Precision.HIGHEST)

    ws, fs = [], []
    for _ in range(_K):
        m = jnp.max(cv, axis=1, keepdims=True)
        eq = cv == m
        cv = jnp.where(eq, neg, cv)
        f = jnp.min(jnp.where(eq, fv, big), axis=1, keepdims=True)
        ws.append(m)
        fs.append(f)

    wv = jnp.concatenate(ws, axis=1)
    fv16 = jnp.concatenate(fs, axis=1)
    e = jnp.exp(wv - wv[:, 0:1])
    wout_ref[...] = e / jnp.sum(e, axis=1, keepdims=True)
    iout_ref[...] = fv16.astype(jnp.int32)


def _select(x2d, w1, b1, *, interpret=False):
    nt = x2d.shape[0]
    T = _SEL_T
    return pl.pallas_call(
        _select_body,
        grid=(nt // T,),
        in_specs=[
            pl.BlockSpec((T, x2d.shape[1]), lambda i: (i, 0)),
            pl.BlockSpec(w1.shape, lambda i: (0, 0)),
            pl.BlockSpec((1, w1.shape[0]), lambda i: (0, 0)),
            pl.BlockSpec((2 * _K + 1, _NC), lambda i: (0, 0)),
        ],
        out_specs=[
            pl.BlockSpec((T, _K), lambda i: (i, 0)),
            pl.BlockSpec((T, _K), lambda i: (i, 0)),
        ],
        out_shape=[
            jax.ShapeDtypeStruct((nt, _K), jnp.float32),
            jax.ShapeDtypeStruct((nt, _K), jnp.int32),
        ],
        interpret=interpret,
    )(x2d, w1, b1.reshape(1, -1), jnp.asarray(_combine_matrix()))


def _sc_combine_body(w_hbm, i_hbm, kb_hbm, out_hbm, idx_v, w_v, rows_a, rows_b,
                     out_a, out_b, gsem_a, gsem_b, ssem_a, ssem_b):
    ntok = out_hbm.shape[0]
    tpw = ntok // _NW
    nchunk = tpw // _CH
    npair = nchunk // 2
    D = kb_hbm.shape[1]
    nj = D // 16
    wid = lax.axis_index("s") * 2 + lax.axis_index("c")
    base_tok = wid * tpw

    pltpu.sync_copy(i_hbm.at[pl.ds(wid * nchunk, nchunk)], idx_v)
    pltpu.sync_copy(w_hbm.at[pl.ds(base_tok * _K, tpw * _K)], w_v)

    def gather(ci, rows, gsem):
        pltpu.async_copy(kb_hbm.at[idx_v.at[ci]], rows, gsem)

    def gather_wait(rows, gsem):
        pltpu.make_async_copy(kb_hbm.at[idx_v.at[0]], rows, gsem).wait()

    def compute_store(ci, rows, out, ssem):
        for t in range(_CH):
            w_row = w_v[pl.ds(ci * _CH * _K + t * _K, _K)]
            wk = [w_row[k] for k in range(_K)]
            for j in range(nj):
                acc = wk[0] * rows[t * _K, pl.ds(j * 16, 16)]
                for k in range(1, _K):
                    acc = acc + wk[k] * rows[t * _K + k, pl.ds(j * 16, 16)]
                out[t, pl.ds(j * 16, 16)] = acc
        pltpu.async_copy(out, out_hbm.at[pl.ds(base_tok + ci * _CH, _CH)], ssem)

    def store_wait(out, ssem):
        pltpu.make_async_copy(out, out_hbm.at[pl.ds(base_tok, _CH)], ssem).wait()

    gather(0, rows_a, gsem_a)
    gather(1, rows_b, gsem_b)

    def pair(p, carry):
        c0 = 2 * p
        gather_wait(rows_a, gsem_a)

        @pl.when(p > 0)
        def _():
            store_wait(out_a, ssem_a)

        compute_store(c0, rows_a, out_a, ssem_a)

        @pl.when(p < npair - 1)
        def _():
            gather(c0 + 2, rows_a, gsem_a)

        gather_wait(rows_b, gsem_b)

        @pl.when(p > 0)
        def _():
            store_wait(out_b, ssem_b)

        compute_store(c0 + 1, rows_b, out_b, ssem_b)

        @pl.when(p < npair - 1)
        def _():
            gather(c0 + 3, rows_b, gsem_b)

        return carry

    lax.fori_loop(0, npair, pair, 0)
    store_wait(out_a, ssem_a)
    store_wait(out_b, ssem_b)


def _combine(w16, i16, kb):
    ntok = w16.shape[0]
    D = kb.shape[1]
    tpw = ntok // _NW
    nchunk = tpw // _CH
    mesh = plsc.VectorSubcoreMesh(core_axis_name="c", subcore_axis_name="s")
    f = pl.kernel(
        _sc_combine_body,
        out_type=jax.ShapeDtypeStruct((ntok, D), jnp.float32),
        mesh=mesh,
        scratch_types=[
            pltpu.VMEM((nchunk, _CH * _K), jnp.int32),
            pltpu.VMEM((tpw * _K,), jnp.float32),
            pltpu.VMEM((_CH * _K, D), jnp.float32),
            pltpu.VMEM((_CH * _K, D), jnp.float32),
            pltpu.VMEM((_CH, D), jnp.float32),
            pltpu.VMEM((_CH, D), jnp.float32),
            pltpu.SemaphoreType.DMA,
            pltpu.SemaphoreType.DMA,
            pltpu.SemaphoreType.DMA,
            pltpu.SemaphoreType.DMA,
        ],
    )
    return f(w16.reshape(-1), i16.reshape(ntok // _CH, _CH * _K), kb)


def _outproj_body(a_ref, w2_ref, b2_ref, y_ref):
    y_ref[...] = lax.dot_general(
        a_ref[...], w2_ref[...], (((1,), (1,)), ((), ())),
        preferred_element_type=jnp.float32) + b2_ref[...]


def _outproj(ans, w2, b2, *, interpret=False):
    nt, D = ans.shape
    E = w2.shape[0]
    T = _OUT_T
    return pl.pallas_call(
        _outproj_body,
        grid=(nt // T,),
        in_specs=[
            pl.BlockSpec((T, D), lambda i: (i, 0)),
            pl.BlockSpec((E, D), lambda i: (0, 0)),
            pl.BlockSpec((1, E), lambda i: (0, 0)),
        ],
        out_specs=pl.BlockSpec((T, E), lambda i: (i, 0)),
        out_shape=jax.ShapeDtypeStruct((nt, E), jnp.float32),
        interpret=interpret,
    )(ans, w2, b2.reshape(1, -1))


def kernel(x, in_proj_w, in_proj_b, out_proj_w, out_proj_b, knowledge_base):
    B, S, E = x.shape
    x2d = x.reshape(B * S, E)
    nsplit = 2
    nh = (B * S) // nsplit
    combined = []
    for i in range(nsplit):
        w16, i16 = _select(x2d[i * nh:(i + 1) * nh], in_proj_w, in_proj_b)
        combined.append(_combine(w16, i16, knowledge_base))
    ys = [_outproj(ans, out_proj_w, out_proj_b) for ans in combined]
    return jnp.concatenate(ys, axis=0).reshape(B, S, E)
